# baseline (device time: 32734 ns/iter reference)
import jax
import jax.numpy as jnp
from jax import lax
from jax.experimental import pallas as pl
from jax.experimental.pallas import tpu as pltpu

B, S, H, Dh, Dr, D = 2, 256, 16, 64, 32, 1024
T = B * S
DC = 64
HH = H // 2
HD = HH * Dh
SCALE = (Dh + Dr) ** -0.5
MESH = pl.DeviceIdType.MESH
BF = jnp.bfloat16
F32 = jnp.float32


def _dot(a, b):
    return jnp.dot(a, b, preferred_element_type=F32)


def kernel(x, Wdkv, Wuk, Wuv, Wq, Wqr, Wkr, Wo):
    def body(x_hbm, wdkv_hbm, wuk_hbm, wuv_hbm, wkr_hbm,
             wq_hbm, wqr_hbm, wo_hbm, out_ref,
             x_v, wdkv_v, wuk_v, wuv_v, wkr_v, wq_v, wqr_v, wo_v,
             c_ref, c_recv, wuk16_ref, wuk_recv, wuv16_ref, wuv_recv,
             o16_ref, out_v, copy_sems, send_sems, recv_sems,
             o_send_sems, o_recv_sems, out_sems):
        my_x = lax.axis_index("x")
        my_y = lax.axis_index("y")
        my_z = lax.axis_index("z")
        xpartner = (1 - my_x, my_y, my_z)
        ypartner = (my_x, 1 - my_y, my_z)

        x_cp = pltpu.make_async_copy(x_hbm, x_v, copy_sems.at[0])
        x_cp.start()

        def make_wq_cps(hy):
            hc0 = hy * HD
            rc0 = hy * HH * Dr
            wq_cp = pltpu.make_async_copy(
                wq_hbm.at[:, hc0:hc0 + HD], wq_v, copy_sems.at[1])
            wqr_cp = pltpu.make_async_copy(
                wqr_hbm.at[:, rc0:rc0 + HH * Dr], wqr_v, copy_sems.at[2])
            return wq_cp, wqr_cp

        @pl.when(my_y == 0)
        def _():
            for cp in make_wq_cps(0):
                cp.start()

        @pl.when(my_y == 1)
        def _():
            for cp in make_wq_cps(1):
                cp.start()

        small_cps = []
        for i, (src, dst) in enumerate([(wdkv_hbm, wdkv_v),
                                        (wuk_hbm, wuk_v),
                                        (wuv_hbm, wuv_v),
                                        (wkr_hbm, wkr_v)]):
            cp = pltpu.make_async_copy(src, dst, copy_sems.at[3 + i])
            cp.start()
            small_cps.append(cp)
        wo_cp = pltpu.make_async_copy(wo_hbm, wo_v, copy_sems.at[7])
        wo_cp.start()

        barrier_sem = pltpu.get_barrier_semaphore()
        pl.semaphore_signal(barrier_sem, inc=1, device_id=xpartner,
                            device_id_type=MESH)
        pl.semaphore_signal(barrier_sem, inc=1, device_id=ypartner,
                            device_id_type=MESH)

        x_cp.wait()
        small_cps[0].wait()
        xf = x_v[...].reshape(T, D)
        x16 = xf.astype(BF)
        c_ref[...] = _dot(x16, wdkv_v[...].astype(BF)).astype(BF)

        def run_half(hy):
            hc0, hc1 = hy * HD, (hy + 1) * HD

            wq_cp, wqr_cp = make_wq_cps(hy)

            small_cps[1].wait()
            small_cps[2].wait()
            wuk16_ref[...] = wuk_v[:, hc0:hc1].astype(BF)
            wuv16_ref[...] = wuv_v[:, hc0:hc1].astype(BF)

            pl.semaphore_wait(barrier_sem, 2)
            rdmas = []
            for i, (src, dst) in enumerate([(c_ref, c_recv),
                                            (wuk16_ref, wuk_recv),
                                            (wuv16_ref, wuv_recv)]):
                r = pltpu.make_async_remote_copy(
                    src_ref=src, dst_ref=dst,
                    send_sem=send_sems.at[i], recv_sem=recv_sems.at[i],
                    device_id=xpartner, device_id_type=MESH,
                )
                r.start()
                rdmas.append(r)

            xq = xf * SCALE
            wq_cp.wait()
            q_all = _dot(xq, wq_v[...])
            wqr_cp.wait()
            qr_all = _dot(xq, wqr_v[...])
            small_cps[3].wait()
            kr_all = _dot(xf, wkr_v[...])

            for r in rdmas:
                r.wait()
            c_loc = c_ref[...]
            c_rem = c_recv[...]
            k_all = _dot(c_loc, wuk16_ref[...]) + _dot(c_rem, wuk_recv[...])
            v_all = _dot(c_loc, wuv16_ref[...]) + _dot(c_rem, wuv_recv[...])

            o_rdmas = []
            for b in range(B):
                kr_b = kr_all[b * S:(b + 1) * S, :].astype(BF)
                for i in range(HH):
                    r0, r1 = b * S, (b + 1) * S
                    lc0, lc1 = i * Dh, (i + 1) * Dh
                    gc0 = hc0 + lc0
                    q = q_all[r0:r1, lc0:lc1].astype(BF)
                    k = k_all[r0:r1, lc0:lc1].astype(BF)
                    v = v_all[r0:r1, lc0:lc1].astype(BF)
                    qr = qr_all[r0:r1, i * Dr:(i + 1) * Dr].astype(BF)
                    s = (lax.dot_general(q, k, (((1,), (1,)), ((), ())),
                                         preferred_element_type=F32)
                         + lax.dot_general(qr, kr_b,
                                           (((1,), (1,)), ((), ())),
                                           preferred_element_type=F32))
                    e = jnp.exp(s.astype(BF))
                    denom = jnp.sum(e, axis=-1, keepdims=True, dtype=F32)
                    o16_ref[r0:r1, gc0:gc0 + Dh] = (_dot(e, v)
                                                    / denom).astype(BF)
                    if i % 2 == 1:
                        jj = b * (HH // 2) + i // 2
                        pc0 = gc0 - Dh
                        r = pltpu.make_async_remote_copy(
                            src_ref=o16_ref.at[r0:r1, pc0:pc0 + 2 * Dh],
                            dst_ref=o16_ref.at[r0:r1, pc0:pc0 + 2 * Dh],
                            send_sem=o_send_sems.at[jj],
                            recv_sem=o_recv_sems.at[jj],
                            device_id=ypartner, device_id_type=MESH,
                        )
                        r.start()
                        o_rdmas.append(r)

            for r in o_rdmas:
                r.wait()

        @pl.when(my_y == 0)
        def _():
            run_half(0)

        @pl.when(my_y == 1)
        def _():
            run_half(1)

        wo_cp.wait()
        out_cps = []
        for b in range(B):
            out_v[b] = _dot(o16_ref[b * S:(b + 1) * S, :], wo_v[...])
            cp = pltpu.make_async_copy(out_v.at[b], out_ref.at[b],
                                       out_sems.at[b])
            cp.start()
            out_cps.append(cp)
        for cp in out_cps:
            cp.wait()

    return pl.pallas_call(
        body,
        out_shape=jax.ShapeDtypeStruct((B, S, D), F32),
        in_specs=[pl.BlockSpec(memory_space=pltpu.HBM)] * 8,
        out_specs=pl.BlockSpec(memory_space=pltpu.HBM),
        scratch_shapes=[
            pltpu.VMEM((B, S, D), F32),
            pltpu.VMEM((D, DC), F32),
            pltpu.VMEM((DC, D), F32),
            pltpu.VMEM((DC, D), F32),
            pltpu.VMEM((D, Dr), F32),
            pltpu.VMEM((D, HD), F32),
            pltpu.VMEM((D, HH * Dr), F32),
            pltpu.VMEM((D, D), F32),
            pltpu.VMEM((T, DC), BF),
            pltpu.VMEM((T, DC), BF),
            pltpu.VMEM((DC, HD), BF),
            pltpu.VMEM((DC, HD), BF),
            pltpu.VMEM((DC, HD), BF),
            pltpu.VMEM((DC, HD), BF),
            pltpu.VMEM((T, D), BF),
            pltpu.VMEM((B, S, D), F32),
            pltpu.SemaphoreType.DMA((8,)),
            pltpu.SemaphoreType.DMA((3,)),
            pltpu.SemaphoreType.DMA((3,)),
            pltpu.SemaphoreType.DMA((HH,)),
            pltpu.SemaphoreType.DMA((HH,)),
            pltpu.SemaphoreType.DMA((B,)),
        ],
        compiler_params=pltpu.CompilerParams(collective_id=0),
    )(x, Wdkv, Wuk, Wuv, Wkr, Wq, Wqr, Wo)


# device time: 29878 ns/iter; 1.0956x vs baseline; 1.0956x over previous
import jax
import jax.numpy as jnp
from jax import lax
from jax.experimental import pallas as pl
from jax.experimental.pallas import tpu as pltpu

B, S, H, Dh, Dr, D = 2, 256, 16, 64, 32, 1024
T = B * S
DC = 64
HH = H // 2
HD = HH * Dh
SCALE = (Dh + Dr) ** -0.5
MESH = pl.DeviceIdType.MESH
BF = jnp.bfloat16
F32 = jnp.float32


def _dot(a, b):
    return jnp.dot(a, b, preferred_element_type=F32)


def kernel(x, Wdkv, Wuk, Wuv, Wq, Wqr, Wkr, Wo):
    def body(x_hbm, wdkv_hbm, wuk_hbm, wuv_hbm, wkr_hbm,
             wq_hbm, wqr_hbm, wo_hbm, out_ref,
             x_v, wdkv_v, wkr_v, wq_v, wqr_v, wo_v,
             c_ref, c_recv, wuk16_ref, wuk_recv, wuv16_ref, wuv_recv,
             o16_ref, out_v, copy_sems, send_sems, recv_sems,
             o_send_sems, o_recv_sems, out_sems):
        my_x = lax.axis_index("x")
        my_y = lax.axis_index("y")
        my_z = lax.axis_index("z")
        xpartner = (1 - my_x, my_y, my_z)
        ypartner = (my_x, 1 - my_y, my_z)

        x_cp = pltpu.make_async_copy(x_hbm, x_v, copy_sems.at[0])
        x_cp.start()

        def make_half_cps(hy):
            hc0 = hy * HD
            rc0 = hy * HH * Dr
            return [
                pltpu.make_async_copy(wq_hbm.at[:, hc0:hc0 + HD], wq_v,
                                      copy_sems.at[1]),
                pltpu.make_async_copy(wqr_hbm.at[:, rc0:rc0 + HH * Dr],
                                      wqr_v, copy_sems.at[2]),
                pltpu.make_async_copy(wuk_hbm.at[:, hc0:hc0 + HD],
                                      wuk16_ref, copy_sems.at[3]),
                pltpu.make_async_copy(wuv_hbm.at[:, hc0:hc0 + HD],
                                      wuv16_ref, copy_sems.at[4]),
            ]

        @pl.when(my_y == 0)
        def _():
            for cp in make_half_cps(0):
                cp.start()

        @pl.when(my_y == 1)
        def _():
            for cp in make_half_cps(1):
                cp.start()

        wdkv_cp = pltpu.make_async_copy(wdkv_hbm, wdkv_v, copy_sems.at[5])
        wdkv_cp.start()
        wkr_cp = pltpu.make_async_copy(wkr_hbm, wkr_v, copy_sems.at[6])
        wkr_cp.start()
        wo_cp = pltpu.make_async_copy(wo_hbm, wo_v, copy_sems.at[7])
        wo_cp.start()

        barrier_sem = pltpu.get_barrier_semaphore()
        pl.semaphore_signal(barrier_sem, inc=1, device_id=xpartner,
                            device_id_type=MESH)
        pl.semaphore_signal(barrier_sem, inc=1, device_id=ypartner,
                            device_id_type=MESH)

        x_cp.wait()
        wdkv_cp.wait()
        xf = x_v[...].reshape(T, D)
        c_ref[...] = _dot(xf, wdkv_v[...]).astype(BF)

        def run_half(hy):
            hc0 = hy * HD
            wq_cp, wqr_cp, wuk_cp, wuv_cp = make_half_cps(hy)

            wuk_cp.wait()
            wuv_cp.wait()
            pl.semaphore_wait(barrier_sem, 2)
            rdmas = []
            for i, (src, dst) in enumerate([(c_ref, c_recv),
                                            (wuk16_ref, wuk_recv),
                                            (wuv16_ref, wuv_recv)]):
                r = pltpu.make_async_remote_copy(
                    src_ref=src, dst_ref=dst,
                    send_sem=send_sems.at[i], recv_sem=recv_sems.at[i],
                    device_id=xpartner, device_id_type=MESH,
                )
                r.start()
                rdmas.append(r)

            xq = xf * jnp.asarray(SCALE, BF)
            wq_cp.wait()
            q_all = _dot(xq, wq_v[...])
            wqr_cp.wait()
            qr_all = _dot(xq, wqr_v[...])
            wkr_cp.wait()
            kr_all = _dot(xf, wkr_v[...])

            for r in rdmas:
                r.wait()
            c_loc = c_ref[...]
            c_rem = c_recv[...]
            k_all = _dot(c_loc, wuk16_ref[...]) + _dot(c_rem, wuk_recv[...])
            v_all = _dot(c_loc, wuv16_ref[...]) + _dot(c_rem, wuv_recv[...])

            o_rdmas = []
            for b in range(B):
                kr_b = kr_all[b * S:(b + 1) * S, :].astype(BF)
                for i in range(HH):
                    r0, r1 = b * S, (b + 1) * S
                    lc0, lc1 = i * Dh, (i + 1) * Dh
                    gc0 = hc0 + lc0
                    q = q_all[r0:r1, lc0:lc1].astype(BF)
                    k = k_all[r0:r1, lc0:lc1].astype(BF)
                    v = v_all[r0:r1, lc0:lc1].astype(BF)
                    qr = qr_all[r0:r1, i * Dr:(i + 1) * Dr].astype(BF)
                    s = (lax.dot_general(q, k, (((1,), (1,)), ((), ())),
                                         preferred_element_type=F32)
                         + lax.dot_general(qr, kr_b,
                                           (((1,), (1,)), ((), ())),
                                           preferred_element_type=F32))
                    e = jnp.exp(s.astype(BF))
                    denom = jnp.sum(e, axis=-1, keepdims=True, dtype=F32)
                    o16_ref[r0:r1, gc0:gc0 + Dh] = (_dot(e, v)
                                                    / denom).astype(BF)
                    if i % 2 == 1:
                        jj = b * (HH // 2) + i // 2
                        pc0 = gc0 - Dh
                        r = pltpu.make_async_remote_copy(
                            src_ref=o16_ref.at[r0:r1, pc0:pc0 + 2 * Dh],
                            dst_ref=o16_ref.at[r0:r1, pc0:pc0 + 2 * Dh],
                            send_sem=o_send_sems.at[jj],
                            recv_sem=o_recv_sems.at[jj],
                            device_id=ypartner, device_id_type=MESH,
                        )
                        r.start()
                        o_rdmas.append(r)

            for r in o_rdmas:
                r.wait()

        @pl.when(my_y == 0)
        def _():
            run_half(0)

        @pl.when(my_y == 1)
        def _():
            run_half(1)

        wo_cp.wait()
        out_cps = []
        for b in range(B):
            out_v[b] = _dot(o16_ref[b * S:(b + 1) * S, :], wo_v[...])
            cp = pltpu.make_async_copy(out_v.at[b], out_ref.at[b],
                                       out_sems.at[b])
            cp.start()
            out_cps.append(cp)
        for cp in out_cps:
            cp.wait()

    call = pl.pallas_call(
        body,
        out_shape=jax.ShapeDtypeStruct((B, S, D), F32),
        in_specs=[pl.BlockSpec(memory_space=pltpu.HBM)] * 8,
        out_specs=pl.BlockSpec(memory_space=pltpu.HBM),
        scratch_shapes=[
            pltpu.VMEM((B, S, D), BF),
            pltpu.VMEM((D, DC), BF),
            pltpu.VMEM((D, Dr), BF),
            pltpu.VMEM((D, HD), BF),
            pltpu.VMEM((D, HH * Dr), BF),
            pltpu.VMEM((D, D), BF),
            pltpu.VMEM((T, DC), BF),
            pltpu.VMEM((T, DC), BF),
            pltpu.VMEM((DC, HD), BF),
            pltpu.VMEM((DC, HD), BF),
            pltpu.VMEM((DC, HD), BF),
            pltpu.VMEM((DC, HD), BF),
            pltpu.VMEM((T, D), BF),
            pltpu.VMEM((B, S, D), F32),
            pltpu.SemaphoreType.DMA((8,)),
            pltpu.SemaphoreType.DMA((3,)),
            pltpu.SemaphoreType.DMA((3,)),
            pltpu.SemaphoreType.DMA((HH,)),
            pltpu.SemaphoreType.DMA((HH,)),
            pltpu.SemaphoreType.DMA((B,)),
        ],
        compiler_params=pltpu.CompilerParams(collective_id=0),
    )
    return call(x.astype(BF), Wdkv.astype(BF), Wuk.astype(BF),
                Wuv.astype(BF), Wkr.astype(BF), Wq.astype(BF),
                Wqr.astype(BF), Wo.astype(BF))


# device time: 28980 ns/iter; 1.1295x vs baseline; 1.0310x over previous
import jax
import jax.numpy as jnp
from jax import lax
from jax.experimental import pallas as pl
from jax.experimental.pallas import tpu as pltpu

B, S, H, Dh, Dr, D = 2, 256, 16, 64, 32, 1024
T = B * S
DC = 64
HH = H // 2
HD = HH * Dh
SCALE = (Dh + Dr) ** -0.5
MESH = pl.DeviceIdType.MESH
BF = jnp.bfloat16
F32 = jnp.float32


def _dot(a, b):
    return jnp.dot(a, b, preferred_element_type=F32)


def kernel(x, Wdkv, Wuk, Wuv, Wq, Wqr, Wkr, Wo):
    def body(x_ref, wdkv_ref, wuk_ref, wuv_ref, wkr_ref,
             wq_ref, wqr_ref, wo_ref, out_ref,
             c_ref, c_recv, wuk_recv, wuv_recv,
             o16_ref, out_v, send_sems, recv_sems,
             o_send_sems, o_recv_sems, out_sems):
        my_x = lax.axis_index("x")
        my_y = lax.axis_index("y")
        my_z = lax.axis_index("z")
        xpartner = (1 - my_x, my_y, my_z)
        ypartner = (my_x, 1 - my_y, my_z)

        barrier_sem = pltpu.get_barrier_semaphore()
        pl.semaphore_signal(barrier_sem, inc=1, device_id=xpartner,
                            device_id_type=MESH)
        pl.semaphore_signal(barrier_sem, inc=1, device_id=ypartner,
                            device_id_type=MESH)

        xf = x_ref[...].reshape(T, D)
        c_ref[...] = _dot(xf, wdkv_ref[...]).astype(BF)

        def run_half(hy):
            hc0 = hy * HD

            pl.semaphore_wait(barrier_sem, 2)
            rdmas = []
            for i, (src, dst) in enumerate(
                    [(c_ref, c_recv),
                     (wuk_ref.at[:, hc0:hc0 + HD], wuk_recv),
                     (wuv_ref.at[:, hc0:hc0 + HD], wuv_recv)]):
                r = pltpu.make_async_remote_copy(
                    src_ref=src, dst_ref=dst,
                    send_sem=send_sems.at[i], recv_sem=recv_sems.at[i],
                    device_id=xpartner, device_id_type=MESH,
                )
                r.start()
                rdmas.append(r)

            xq = xf * jnp.asarray(SCALE, BF)
            q_all = _dot(xq, wq_ref[:, hc0:hc0 + HD])
            rc0 = hy * HH * Dr
            qr_all = _dot(xq, wqr_ref[:, rc0:rc0 + HH * Dr])
            kr_all = _dot(xf, wkr_ref[...])

            for r in rdmas:
                r.wait()
            c_loc = c_ref[...]
            c_rem = c_recv[...]
            wuk_loc = wuk_ref[:, hc0:hc0 + HD]
            wuv_loc = wuv_ref[:, hc0:hc0 + HD]
            k_all = _dot(c_loc, wuk_loc) + _dot(c_rem, wuk_recv[...])
            v_all = _dot(c_loc, wuv_loc) + _dot(c_rem, wuv_recv[...])

            o_rdmas = []
            for b in range(B):
                kr_b = kr_all[b * S:(b + 1) * S, :].astype(BF)
                for i in range(HH):
                    r0, r1 = b * S, (b + 1) * S
                    lc0, lc1 = i * Dh, (i + 1) * Dh
                    gc0 = hc0 + lc0
                    q = q_all[r0:r1, lc0:lc1].astype(BF)
                    k = k_all[r0:r1, lc0:lc1].astype(BF)
                    v = v_all[r0:r1, lc0:lc1].astype(BF)
                    qr = qr_all[r0:r1, i * Dr:(i + 1) * Dr].astype(BF)
                    s = (lax.dot_general(q, k, (((1,), (1,)), ((), ())),
                                         preferred_element_type=F32)
                         + lax.dot_general(qr, kr_b,
                                           (((1,), (1,)), ((), ())),
                                           preferred_element_type=F32))
                    e = jnp.exp(s.astype(BF))
                    denom = jnp.sum(e, axis=-1, keepdims=True, dtype=F32)
                    o16_ref[r0:r1, gc0:gc0 + Dh] = (_dot(e, v)
                                                    / denom).astype(BF)
                    if i % 2 == 1:
                        jj = b * (HH // 2) + i // 2
                        pc0 = gc0 - Dh
                        r = pltpu.make_async_remote_copy(
                            src_ref=o16_ref.at[r0:r1, pc0:pc0 + 2 * Dh],
                            dst_ref=o16_ref.at[r0:r1, pc0:pc0 + 2 * Dh],
                            send_sem=o_send_sems.at[jj],
                            recv_sem=o_recv_sems.at[jj],
                            device_id=ypartner, device_id_type=MESH,
                        )
                        r.start()
                        o_rdmas.append(r)

            for r in o_rdmas:
                r.wait()

        @pl.when(my_y == 0)
        def _():
            run_half(0)

        @pl.when(my_y == 1)
        def _():
            run_half(1)

        out_cps = []
        for b in range(B):
            out_v[b] = _dot(o16_ref[b * S:(b + 1) * S, :],
                            wo_ref[...]).astype(BF)
            cp = pltpu.make_async_copy(out_v.at[b], out_ref.at[b],
                                       out_sems.at[b])
            cp.start()
            out_cps.append(cp)
        for cp in out_cps:
            cp.wait()

    call = pl.pallas_call(
        body,
        out_shape=jax.ShapeDtypeStruct((B, S, D), BF),
        in_specs=[pl.BlockSpec(memory_space=pltpu.VMEM)] * 8,
        out_specs=pl.BlockSpec(memory_space=pltpu.HBM),
        scratch_shapes=[
            pltpu.VMEM((T, DC), BF),
            pltpu.VMEM((T, DC), BF),
            pltpu.VMEM((DC, HD), BF),
            pltpu.VMEM((DC, HD), BF),
            pltpu.VMEM((T, D), BF),
            pltpu.VMEM((B, S, D), BF),
            pltpu.SemaphoreType.DMA((3,)),
            pltpu.SemaphoreType.DMA((3,)),
            pltpu.SemaphoreType.DMA((HH,)),
            pltpu.SemaphoreType.DMA((HH,)),
            pltpu.SemaphoreType.DMA((B,)),
        ],
        compiler_params=pltpu.CompilerParams(collective_id=0),
    )
    return call(x.astype(BF), Wdkv.astype(BF), Wuk.astype(BF),
                Wuv.astype(BF), Wkr.astype(BF), Wq.astype(BF),
                Wqr.astype(BF), Wo.astype(BF))
